# Initial kernel scaffold; baseline (speedup 1.0000x reference)
import jax
import jax.numpy as jnp
from jax.experimental import pallas as pl


def kernel(clip_id, scene_face, scene_pose, scene_presence, scene_size, scene_flip, scene_x_center, scene_y_center, scene_area, scene_width, scene_height, clip_table, face_table, pose_table, presence_table, size_table, flip_table, pos_W, pos_b):
    raise NotImplementedError("write your pallas kernel here")


# speedup vs baseline: 3.3273x; 3.3273x over previous
import functools

import jax
import jax.numpy as jnp
from jax import lax
from jax.experimental import pallas as pl
from jax.experimental.pallas import tpu as pltpu
from jax.experimental.pallas import tpu_sc as plsc

_B, _L = 4096, 50
_N = _B * _L
_D = 64
_NW = 32
_T = _N // _NW
_C = 800
_NCHUNK = _T // _C

_OUT_D = 164


def _sc_gather(table, idx):
    mesh = plsc.VectorSubcoreMesh(core_axis_name="c", subcore_axis_name="s")

    @functools.partial(
        pl.kernel,
        mesh=mesh,
        compiler_params=pltpu.CompilerParams(use_tc_tiling_on_sc=False),
        out_type=jax.ShapeDtypeStruct((_N, _D), jnp.float32),
        scratch_types=[
            pltpu.VMEM((_C,), jnp.int32),
            pltpu.VMEM((_C, _D), jnp.float32),
            pltpu.SemaphoreType.DMA,
        ],
    )
    def k(table_hbm, idx_hbm, out_hbm, idx_v, rows_v, sem):
        wid = lax.axis_index("s") * 2 + lax.axis_index("c")
        base0 = wid * _T
        for j in range(_NCHUNK):
            base = base0 + j * _C
            pltpu.sync_copy(idx_hbm.at[pl.ds(base, _C)], idx_v)
            pltpu.async_copy(table_hbm.at[idx_v], rows_v, sem).wait()
            pltpu.sync_copy(rows_v, out_hbm.at[pl.ds(base, _C)])

    return k(table, idx)


_BT = 1024


def _tc_assemble(rows, fid, pid, prid, sid, flid, xc, yc, ar, wd, ht,
                 face_t, pose_t, pres_t, size_t, flip_t, pos_W, pos_b):
    grid = (_N // _BT,)

    def body(rows_ref, fid_ref, pid_ref, prid_ref, sid_ref, flid_ref,
             xc_ref, yc_ref, ar_ref, wd_ref, ht_ref,
             face_ref, pose_ref, pres_ref, size_ref, flip_ref,
             W_ref, b_ref, out_ref):
        out_ref[:, 0:64] = rows_ref[...]

        def lut(ids_ref, tab_ref, nrows):
            idx = ids_ref[...]
            acc = jnp.where(idx == 0, 1.0, 0.0) * tab_ref[0:1, :]
            for r in range(1, nrows):
                acc = acc + jnp.where(idx == r, 1.0, 0.0) * tab_ref[r:r + 1, :]
            return acc

        out_ref[:, 64:84] = lut(pid_ref, pose_ref, 7)
        out_ref[:, 84:104] = lut(fid_ref, face_ref, 5)
        out_ref[:, 104:114] = lut(prid_ref, pres_ref, 2)
        pos = (xc_ref[...] * W_ref[0:1, :] + yc_ref[...] * W_ref[1:2, :]
               + ar_ref[...] * W_ref[2:3, :] + wd_ref[...] * W_ref[3:4, :]
               + ht_ref[...] * W_ref[4:5, :] + b_ref[0:1, :])
        out_ref[:, 114:144] = pos
        out_ref[:, 144:154] = lut(sid_ref, size_ref, 3)
        out_ref[:, 154:164] = lut(flid_ref, flip_ref, 2)

    tok_spec = pl.BlockSpec((_BT, 1), lambda i: (i, 0))
    full = lambda shp: pl.BlockSpec(shp, lambda i: (0, 0))
    return pl.pallas_call(
        body,
        grid=grid,
        in_specs=[
            pl.BlockSpec((_BT, _D), lambda i: (i, 0)),
            tok_spec, tok_spec, tok_spec, tok_spec, tok_spec,
            tok_spec, tok_spec, tok_spec, tok_spec, tok_spec,
            full((5, 20)), full((7, 20)), full((2, 10)), full((3, 10)),
            full((2, 10)), full((5, 30)), full((1, 30)),
        ],
        out_specs=pl.BlockSpec((_BT, _OUT_D), lambda i: (i, 0)),
        out_shape=jax.ShapeDtypeStruct((_N, _OUT_D), jnp.float32),
        compiler_params=pltpu.CompilerParams(
            dimension_semantics=("arbitrary",),
        ),
    )(rows, fid, pid, prid, sid, flid, xc, yc, ar, wd, ht,
      face_t, pose_t, pres_t, size_t, flip_t, pos_W, pos_b)


def kernel(clip_id, scene_face, scene_pose, scene_presence, scene_size,
           scene_flip, scene_x_center, scene_y_center, scene_area,
           scene_width, scene_height, clip_table, face_table, pose_table,
           presence_table, size_table, flip_table, pos_W, pos_b):
    col = lambda a: a.reshape(_N, 1)
    rows = _sc_gather(clip_table, clip_id.reshape(_N).astype(jnp.int32))
    out = _tc_assemble(
        rows,
        col(scene_face.astype(jnp.int32)), col(scene_pose.astype(jnp.int32)),
        col(scene_presence.astype(jnp.int32)), col(scene_size.astype(jnp.int32)),
        col(scene_flip.astype(jnp.int32)),
        col(scene_x_center), col(scene_y_center), col(scene_area),
        col(scene_width), col(scene_height),
        face_table, pose_table, presence_table, size_table, flip_table,
        pos_W, pos_b.reshape(1, 30))
    return out.reshape(_B, _L, _OUT_D)


# speedup vs baseline: 22.7809x; 6.8467x over previous
import functools

import jax
import jax.numpy as jnp
from jax import lax
from jax.experimental import pallas as pl
from jax.experimental.pallas import tpu as pltpu
from jax.experimental.pallas import tpu_sc as plsc

_B, _L = 4096, 50
_N = _B * _L
_D = 64
_NW = 32
_T = _N // _NW
_C = 800
_NCHUNK = _T // _C

_OUT_D = 164
_BBL = 256


def _sc_gather(table, idx):
    mesh = plsc.VectorSubcoreMesh(core_axis_name="c", subcore_axis_name="s")

    @functools.partial(
        pl.kernel,
        mesh=mesh,
        compiler_params=pltpu.CompilerParams(use_tc_tiling_on_sc=False),
        out_type=jax.ShapeDtypeStruct((_N, _D), jnp.float32),
        scratch_types=[
            pltpu.VMEM((_C,), jnp.int32),
            pltpu.VMEM((_C, _D), jnp.float32),
            pltpu.SemaphoreType.DMA,
        ],
    )
    def k(table_hbm, idx_hbm, out_hbm, idx_v, rows_v, sem):
        wid = lax.axis_index("s") * 2 + lax.axis_index("c")
        base0 = wid * _T
        for j in range(_NCHUNK):
            base = base0 + j * _C
            pltpu.sync_copy(idx_hbm.at[pl.ds(base, _C)], idx_v)
            pltpu.async_copy(table_hbm.at[idx_v], rows_v, sem).wait()
            pltpu.sync_copy(rows_v, out_hbm.at[pl.ds(base, _C)])

    return k(table, idx)


def _tc_assemble(clips3, idsf, bigT):
    grid = (_B // _BBL,)

    def body(clips_ref, fid_ref, pid_ref, prid_ref, sid_ref, flid_ref,
             xc_ref, yc_ref, ar_ref, wd_ref, ht_ref, bigT_ref, out_ref):
        BT = bigT_ref[...]
        it = lax.broadcasted_iota(jnp.int32, (32, _BBL), 0)
        one = jnp.ones((32, _BBL), jnp.float32)
        zero = jnp.zeros((32, _BBL), jnp.float32)
        for l in range(_L):
            x = clips_ref[l, :, :]
            xT = x.T
            out_ref[l, 0:_D, 0:128] = xT[0:_D, :]
            out_ref[l, 0:_D, 128:256] = xT[_D:128, :]

            pose = pid_ref[l:l + 1, :]
            face = fid_ref[l:l + 1, :]
            pres = prid_ref[l:l + 1, :]
            size = sid_ref[l:l + 1, :]
            flip = flid_ref[l:l + 1, :]
            F = jnp.where(
                it < 7, jnp.where(pose == it, one, zero),
                jnp.where(
                    it < 12, jnp.where(face == it - 7, one, zero),
                    jnp.where(
                        it < 14, jnp.where(pres == it - 12, one, zero),
                        jnp.where(
                            it == 14, xc_ref[l:l + 1, :],
                            jnp.where(
                                it == 15, yc_ref[l:l + 1, :],
                                jnp.where(
                                    it == 16, ar_ref[l:l + 1, :],
                                    jnp.where(
                                        it == 17, wd_ref[l:l + 1, :],
                                        jnp.where(
                                            it == 18, ht_ref[l:l + 1, :],
                                            jnp.where(
                                                it == 19, one,
                                                jnp.where(
                                                    it < 23,
                                                    jnp.where(size == it - 20, one, zero),
                                                    jnp.where(
                                                        it < 25,
                                                        jnp.where(flip == it - 23, one, zero),
                                                        zero)))))))))))
            o100 = jnp.dot(BT, F, preferred_element_type=jnp.float32)
            out_ref[l, _D:_OUT_D, :] = o100

    id_spec = pl.BlockSpec((_L, _BBL), lambda i: (0, i))
    return pl.pallas_call(
        body,
        grid=grid,
        in_specs=[
            pl.BlockSpec((_L, _BBL // 2, 128), lambda i: (0, i, 0)),
            id_spec, id_spec, id_spec, id_spec, id_spec,
            id_spec, id_spec, id_spec, id_spec, id_spec,
            pl.BlockSpec((100, 32), lambda i: (0, 0)),
        ],
        out_specs=pl.BlockSpec((_L, _OUT_D, _BBL), lambda i: (0, 0, i)),
        out_shape=jax.ShapeDtypeStruct((_L, _OUT_D, _B), jnp.float32),
        compiler_params=pltpu.CompilerParams(
            dimension_semantics=("arbitrary",),
        ),
    )(clips3, *idsf, bigT)


def kernel(clip_id, scene_face, scene_pose, scene_presence, scene_size,
           scene_flip, scene_x_center, scene_y_center, scene_area,
           scene_width, scene_height, clip_table, face_table, pose_table,
           presence_table, size_table, flip_table, pos_W, pos_b):
    idxT = (jnp.transpose(clip_id).astype(jnp.int32)
            .reshape(_L, _B // 256, 2, 128)
            .transpose(0, 1, 3, 2).reshape(_N))
    rows = _sc_gather(clip_table, idxT)
    clips3 = rows.reshape(_L, _B // 2, 128)

    Z = jnp.zeros((100, 32), jnp.float32)
    Z = Z.at[0:20, 0:7].set(pose_table.T)
    Z = Z.at[20:40, 7:12].set(face_table.T)
    Z = Z.at[40:50, 12:14].set(presence_table.T)
    Z = Z.at[50:80, 14:19].set(pos_W.T)
    Z = Z.at[50:80, 19].set(pos_b)
    Z = Z.at[80:90, 20:23].set(size_table.T)
    Z = Z.at[90:100, 23:25].set(flip_table.T)

    tr = lambda a: jnp.transpose(a)
    idsf = [tr(scene_face.astype(jnp.int32)), tr(scene_pose.astype(jnp.int32)),
            tr(scene_presence.astype(jnp.int32)), tr(scene_size.astype(jnp.int32)),
            tr(scene_flip.astype(jnp.int32)),
            tr(scene_x_center), tr(scene_y_center), tr(scene_area),
            tr(scene_width), tr(scene_height)]
    outT = _tc_assemble(clips3, idsf, Z)
    return jnp.transpose(outT, (2, 0, 1))
